# Initial kernel scaffold; baseline (speedup 1.0000x reference)
import jax
import jax.numpy as jnp
from jax.experimental import pallas as pl


def kernel(x, edge_attr, tree_alpha, W_i, W_h, W_o, b_o, edge_index, tree_tgt_nodes, graph_ids):
    raise NotImplementedError("write your pallas kernel here")


# speedup vs baseline: 1.9703x; 1.9703x over previous
import functools

import jax
import jax.numpy as jnp
from jax import lax
from jax.experimental import pallas as pl
from jax.experimental.pallas import tpu as pltpu
from jax.experimental.pallas import tpu_sc as plsc

N = 10000
E = 160000
EH = E // 2
A = 128
B = 16
H = 512
DEPTH = 6
T = 40000
G = 100
GP = 104

NC = 2
NS = 16
COLS = 128
NCOL = H // COLS

_mesh = plsc.VectorSubcoreMesh(core_axis_name="c", subcore_axis_name="s")

_f32 = jnp.float32


def _dotT(a, w):
    return lax.dot_general(a, w, (((1,), (1,)), ((), ())),
                           preferred_element_type=_f32)


def _make_scatter(M, CH):
    n_chunks = M // CH
    RCH = 80
    n_rchunks = N // RCH

    @functools.partial(
        pl.kernel,
        out_type=jax.ShapeDtypeStruct((N, H), _f32),
        mesh=_mesh,
        scratch_types=[
            pltpu.VMEM((CH,), jnp.int32),
            pltpu.VMEM((CH, COLS), _f32),
            pltpu.VMEM_SHARED((N, COLS), _f32),
        ],
    )
    def scat(data, idx, init, out, idx_v, data_v, table):
        c = lax.axis_index("c")
        s = lax.axis_index("s")
        for j in range(NCOL // NC):
            col = (c + NC * j) * COLS
            @pl.loop(s, n_rchunks, step=NS)
            def _(r):
                r0 = r * RCH
                pltpu.sync_copy(
                    init.at[pl.ds(r0, RCH), pl.ds(col, COLS)],
                    table.at[pl.ds(r0, RCH)])

            plsc.subcore_barrier()

            @pl.loop(s, n_chunks, step=NS)
            def _(k):
                e0 = k * CH
                pltpu.sync_copy(idx.at[pl.ds(e0, CH)], idx_v)
                pltpu.sync_copy(data.at[pl.ds(e0, CH), pl.ds(col, COLS)],
                                data_v)
                pltpu.sync_copy(data_v, table.at[idx_v], add=True)

            plsc.subcore_barrier()

            @pl.loop(s, n_rchunks, step=NS)
            def _(r):
                r0 = r * RCH
                pltpu.sync_copy(
                    table.at[pl.ds(r0, RCH)],
                    out.at[pl.ds(r0, RCH), pl.ds(col, COLS)])

            plsc.subcore_barrier()

    return scat


def _make_gather(M, CH=128):
    n_chunks = M // CH

    @functools.partial(
        pl.kernel,
        out_type=jax.ShapeDtypeStruct((M, H), _f32),
        mesh=_mesh,
        scratch_types=[
            pltpu.VMEM((CH,), jnp.int32),
            pltpu.VMEM((CH, H), _f32),
            pltpu.SemaphoreType.DMA,
        ],
    )
    def gat(table, idx, out, idx_v, rows_v, sem):
        c = lax.axis_index("c")
        s = lax.axis_index("s")
        w = s * NC + c

        @pl.loop(w, n_chunks, step=NC * NS)
        def _(k):
            e0 = k * CH
            pltpu.sync_copy(idx.at[pl.ds(e0, CH)], idx_v)
            pltpu.async_copy(table.at[idx_v], rows_v, sem).wait()
            pltpu.sync_copy(rows_v, out.at[pl.ds(e0, CH)])

    return gat


_scatter_E = _make_scatter(E, 128)
_scatter_T = _make_scatter(T, 64)
_gather_E = _make_gather(E)


BLKE = 800
BLKN = 1000


def _k0_body(x_b, wix, out):
    out[...] = _dotT(x_b[...], wix[...])


def _tc_xw(x, wix):
    return pl.pallas_call(
        _k0_body,
        grid=(N // BLKN,),
        in_specs=[
            pl.BlockSpec((BLKN, A), lambda i: (i, 0)),
            pl.BlockSpec((H, A), lambda i: (0, 0)),
        ],
        out_specs=pl.BlockSpec((BLKN, H), lambda i: (i, 0)),
        out_shape=jax.ShapeDtypeStruct((N, H), _f32),
    )(x, wix)


def _k1_body(g0, ea, wie, mi_o, msg_o):
    for d in range(2):
        v = g0[d] + _dotT(ea[d], wie[...])
        mi_o[d] = v
        msg_o[d] = jnp.maximum(v, 0.0)


def _tc_init(g0, ea, wie):
    return pl.pallas_call(
        _k1_body,
        grid=(EH // BLKE,),
        in_specs=[
            pl.BlockSpec((2, BLKE, H), lambda i: (0, i, 0)),
            pl.BlockSpec((2, BLKE, B), lambda i: (0, i, 0)),
            pl.BlockSpec((H, B), lambda i: (0, 0)),
        ],
        out_specs=[
            pl.BlockSpec((2, BLKE, H), lambda i: (0, i, 0)),
            pl.BlockSpec((2, BLKE, H), lambda i: (0, i, 0)),
        ],
        out_shape=[
            jax.ShapeDtypeStruct((2, EH, H), _f32),
            jax.ShapeDtypeStruct((2, EH, H), _f32),
        ],
    )(g0, ea, wie)


def _k2_body(mi, g, msg, wh, out):
    for d in range(2):
        acc = g[d] - msg[1 - d]
        out[d] = jnp.maximum(mi[d] + _dotT(acc, wh[...]), 0.0)


def _tc_step(mi, g, msg, wh):
    return pl.pallas_call(
        _k2_body,
        grid=(EH // BLKE,),
        in_specs=[
            pl.BlockSpec((2, BLKE, H), lambda i: (0, i, 0)),
            pl.BlockSpec((2, BLKE, H), lambda i: (0, i, 0)),
            pl.BlockSpec((2, BLKE, H), lambda i: (0, i, 0)),
            pl.BlockSpec((H, H), lambda i: (0, 0)),
        ],
        out_specs=pl.BlockSpec((2, BLKE, H), lambda i: (0, i, 0)),
        out_shape=jax.ShapeDtypeStruct((2, EH, H), _f32),
    )(mi, g, msg, wh)


def _k3_body(x_b, m_b, gid_b, wox, wom, bo, out, acc, cnt):
    i = pl.program_id(0)
    nsteps = pl.num_programs(0)
    h = jnp.maximum(
        _dotT(x_b[...], wox[...]) + _dotT(m_b[...], wom[...]) + bo[...], 0.0)
    ids = gid_b[...]
    cols = lax.broadcasted_iota(jnp.int32, (1, GP), 1)
    oh = (ids == cols).astype(_f32)
    contrib = lax.dot_general(oh, h, (((0,), (0,)), ((), ())),
                              preferred_element_type=_f32)
    ones = jnp.ones((BLKN, 1), _f32)
    ccol = lax.dot_general(oh, ones, (((0,), (0,)), ((), ())),
                           preferred_element_type=_f32)

    @pl.when(i == 0)
    def _():
        acc[...] = contrib
        cnt[...] = ccol

    @pl.when(i > 0)
    def _():
        acc[...] += contrib
        cnt[...] += ccol

    @pl.when(i == nsteps - 1)
    def _():
        out[...] = acc[...] / jnp.maximum(cnt[...], 1.0)


def _tc_readout(x, m, gid, wox, wom, bo):
    return pl.pallas_call(
        _k3_body,
        grid=(N // BLKN,),
        in_specs=[
            pl.BlockSpec((BLKN, A), lambda i: (i, 0)),
            pl.BlockSpec((BLKN, H), lambda i: (i, 0)),
            pl.BlockSpec((BLKN, 1), lambda i: (i, 0)),
            pl.BlockSpec((H, A), lambda i: (0, 0)),
            pl.BlockSpec((H, H), lambda i: (0, 0)),
            pl.BlockSpec((1, H), lambda i: (0, 0)),
        ],
        out_specs=pl.BlockSpec((GP, H), lambda i: (0, 0)),
        out_shape=jax.ShapeDtypeStruct((GP, H), _f32),
        scratch_shapes=[
            pltpu.VMEM((GP, H), _f32),
            pltpu.VMEM((GP, 1), _f32),
        ],
    )(x, m, gid, wox, wom, bo)


def kernel(x, edge_attr, tree_alpha, W_i, W_h, W_o, b_o, edge_index,
           tree_tgt_nodes, graph_ids):
    src = edge_index[0].astype(jnp.int32)
    dst = edge_index[1].astype(jnp.int32)
    src2 = jnp.concatenate([src[0::2], src[1::2]])
    dst2 = jnp.concatenate([dst[0::2], dst[1::2]])
    ea2 = jnp.stack([edge_attr[0::2], edge_attr[1::2]])
    tt = tree_tgt_nodes.astype(jnp.int32)
    gid = graph_ids.astype(jnp.int32).reshape(N, 1)
    wix = W_i[:, :A]
    wie = W_i[:, A:]
    wox = W_o[:, :A]
    wom = W_o[:, A:]
    bo = b_o.reshape(1, H)

    zero_init = jnp.zeros((N, H), _f32)
    node_alpha = _scatter_T(tree_alpha, tt, zero_init)
    xw = _tc_xw(x, wix)
    g0 = _gather_E(xw, src2).reshape(2, EH, H)
    mi, msg = _tc_init(g0, ea2, wie)
    for _ in range(DEPTH - 1):
        s_tab = _scatter_E(msg.reshape(E, H), dst2, node_alpha)
        g = _gather_E(s_tab, src2).reshape(2, EH, H)
        msg = _tc_step(mi, g, msg, W_h)
    m = _scatter_E(msg.reshape(E, H), dst2, node_alpha)
    gr = _tc_readout(x, m, gid, wox, wom, bo)
    return gr[:G]


# speedup vs baseline: 2.4352x; 1.2360x over previous
import functools

import jax
import jax.numpy as jnp
from jax import lax
from jax.experimental import pallas as pl
from jax.experimental.pallas import tpu as pltpu
from jax.experimental.pallas import tpu_sc as plsc

N = 10000
E = 160000
EH = E // 2
A = 128
B = 16
H = 512
DEPTH = 6
T = 40000
G = 100
GP = 104

NC = 2
NS = 16
COLS = 128
NCOL = H // COLS

_mesh = plsc.VectorSubcoreMesh(core_axis_name="c", subcore_axis_name="s")

_f32 = jnp.float32


def _dotT(a, w):
    return lax.dot_general(a, w, (((1,), (1,)), ((), ())),
                           preferred_element_type=_f32)


def _make_scatter(M, CH):
    n_chunks = M // CH
    RCH = 80
    n_rchunks = N // RCH

    @functools.partial(
        pl.kernel,
        out_type=jax.ShapeDtypeStruct((N, H), _f32),
        mesh=_mesh,
        scratch_types=[
            pltpu.VMEM((CH,), jnp.int32),
            pltpu.VMEM((CH, COLS), _f32),
            pltpu.VMEM_SHARED((N, COLS), _f32),
        ],
    )
    def scat(data, idx, init, out, idx_v, data_v, table):
        c = lax.axis_index("c")
        s = lax.axis_index("s")
        for j in range(NCOL // NC):
            col = (c + NC * j) * COLS
            @pl.loop(s, n_rchunks, step=NS)
            def _(r):
                r0 = r * RCH
                pltpu.sync_copy(
                    init.at[pl.ds(r0, RCH), pl.ds(col, COLS)],
                    table.at[pl.ds(r0, RCH)])

            plsc.subcore_barrier()

            @pl.loop(s, n_chunks, step=NS)
            def _(k):
                e0 = k * CH
                pltpu.sync_copy(idx.at[pl.ds(e0, CH)], idx_v)
                pltpu.sync_copy(data.at[pl.ds(e0, CH), pl.ds(col, COLS)],
                                data_v)
                pltpu.sync_copy(data_v, table.at[idx_v], add=True)

            plsc.subcore_barrier()

            @pl.loop(s, n_rchunks, step=NS)
            def _(r):
                r0 = r * RCH
                pltpu.sync_copy(
                    table.at[pl.ds(r0, RCH)],
                    out.at[pl.ds(r0, RCH), pl.ds(col, COLS)])

            plsc.subcore_barrier()

    return scat


def _make_gather(M, CH=128):
    n_chunks = M // CH

    @functools.partial(
        pl.kernel,
        out_type=jax.ShapeDtypeStruct((M, H), _f32),
        mesh=_mesh,
        scratch_types=[
            pltpu.VMEM((CH,), jnp.int32),
            pltpu.VMEM((CH, H), _f32),
            pltpu.SemaphoreType.DMA,
        ],
    )
    def gat(table, idx, out, idx_v, rows_v, sem):
        c = lax.axis_index("c")
        s = lax.axis_index("s")
        w = s * NC + c

        @pl.loop(w, n_chunks, step=NC * NS)
        def _(k):
            e0 = k * CH
            pltpu.sync_copy(idx.at[pl.ds(e0, CH)], idx_v)
            pltpu.async_copy(table.at[idx_v], rows_v, sem).wait()
            pltpu.sync_copy(rows_v, out.at[pl.ds(e0, CH)])

    return gat


_scatter_E = _make_scatter(E, 128)
_scatter_T = _make_scatter(T, 64)
_gather_E = _make_gather(E)


BCH = 128
NCHF = E // BCH
NTF = 80


@functools.partial(
    pl.kernel,
    out_type=jax.ShapeDtypeStruct((E, H), _f32),
    mesh=_mesh,
    scratch_types=[
        pltpu.VMEM((128,), jnp.int32),
        pltpu.VMEM((128,), jnp.int32),
        pltpu.VMEM((BCH, COLS), _f32),
        pltpu.VMEM((BCH, COLS), _f32),
        pltpu.SemaphoreType.DMA,
        pltpu.SemaphoreType.DMA,
        pltpu.SemaphoreType.DMA,
        pltpu.SemaphoreType.DMA,
        pltpu.VMEM_SHARED((N, COLS), _f32),
    ],
)
def _fused_step(msg, dsti, srci, init, gout,
                ia, ib, dat0, dat1, ds0, ds1, ss0, ss1, table):
    c = lax.axis_index("c")
    s = lax.axis_index("s")
    idxs = (ia, ib)
    dats = (dat0, dat1)
    dsems = (ds0, ds1)
    ssems = (ss0, ss1)

    for j in range(NCOL // NC):
        col = (c + NC * j) * COLS

        @pl.loop(s, N // 80, step=NS)
        def _(r):
            pltpu.sync_copy(init.at[pl.ds(r * 80, 80), pl.ds(col, COLS)],
                            table.at[pl.ds(r * 80, 80)])

        plsc.subcore_barrier()

        def slot_a(t, par):
            b = s + t * NS

            @pl.when(jnp.logical_and(t >= 2, (b - 2 * NS) < NCHF))
            def _():
                pltpu.make_async_copy(dats[par], table.at[idxs[par]],
                                      ssems[par]).wait()

            @pl.when(b < NCHF)
            def _():
                cp = pltpu.async_copy(
                    msg.at[pl.ds(b * BCH, BCH), pl.ds(col, COLS)],
                    dats[par], dsems[par])
                pltpu.sync_copy(dsti.at[pl.ds(b * BCH, BCH)], idxs[par])
                cp.wait()
                pltpu.async_copy(dats[par], table.at[idxs[par]],
                                 ssems[par], add=True)

        @pl.loop(0, NTF // 2)
        def _(tp):
            slot_a(2 * tp, 0)
            slot_a(2 * tp + 1, 1)

        for t_e, par in ((NTF - 2, 0), (NTF - 1, 1)):
            @pl.when((s + t_e * NS) < NCHF)
            def _():
                pltpu.make_async_copy(dats[par], table.at[idxs[par]],
                                      ssems[par]).wait()

        plsc.subcore_barrier()

        def slot_b(t, par):
            b = s + t * NS

            @pl.when(jnp.logical_and(t >= 2, (b - 2 * NS) < NCHF))
            def _():
                pltpu.make_async_copy(
                    dats[par],
                    gout.at[pl.ds((b - 2 * NS) * BCH, BCH), pl.ds(col, COLS)],
                    ssems[par]).wait()

            @pl.when(b < NCHF)
            def _():
                pltpu.sync_copy(srci.at[pl.ds(b * BCH, BCH)], idxs[par])
                pltpu.async_copy(table.at[idxs[par]], dats[par],
                                 dsems[par]).wait()
                pltpu.async_copy(
                    dats[par],
                    gout.at[pl.ds(b * BCH, BCH), pl.ds(col, COLS)],
                    ssems[par])

        @pl.loop(0, NTF // 2)
        def _(tp):
            slot_b(2 * tp, 0)
            slot_b(2 * tp + 1, 1)

        for t_e, par in ((NTF - 2, 0), (NTF - 1, 1)):
            b_e = s + t_e * NS

            @pl.when(b_e < NCHF)
            def _():
                pltpu.make_async_copy(
                    dats[par],
                    gout.at[pl.ds(b_e * BCH, BCH), pl.ds(col, COLS)],
                    ssems[par]).wait()

        plsc.subcore_barrier()


BLKE = 800
BLKN = 1000


def _k0_body(x_b, wix, out):
    out[...] = _dotT(x_b[...], wix[...])


def _tc_xw(x, wix):
    return pl.pallas_call(
        _k0_body,
        grid=(N // BLKN,),
        in_specs=[
            pl.BlockSpec((BLKN, A), lambda i: (i, 0)),
            pl.BlockSpec((H, A), lambda i: (0, 0)),
        ],
        out_specs=pl.BlockSpec((BLKN, H), lambda i: (i, 0)),
        out_shape=jax.ShapeDtypeStruct((N, H), _f32),
    )(x, wix)


def _k1_body(g0, ea, wie, mi_o, msg_o):
    for d in range(2):
        v = g0[d] + _dotT(ea[d], wie[...])
        mi_o[d] = v
        msg_o[d] = jnp.maximum(v, 0.0)


def _tc_init(g0, ea, wie):
    return pl.pallas_call(
        _k1_body,
        grid=(EH // BLKE,),
        in_specs=[
            pl.BlockSpec((2, BLKE, H), lambda i: (0, i, 0)),
            pl.BlockSpec((2, BLKE, B), lambda i: (0, i, 0)),
            pl.BlockSpec((H, B), lambda i: (0, 0)),
        ],
        out_specs=[
            pl.BlockSpec((2, BLKE, H), lambda i: (0, i, 0)),
            pl.BlockSpec((2, BLKE, H), lambda i: (0, i, 0)),
        ],
        out_shape=[
            jax.ShapeDtypeStruct((2, EH, H), _f32),
            jax.ShapeDtypeStruct((2, EH, H), _f32),
        ],
    )(g0, ea, wie)


def _k2_body(mi, g, msg, wh, out):
    for d in range(2):
        acc = g[d] - msg[1 - d]
        out[d] = jnp.maximum(mi[d] + _dotT(acc, wh[...]), 0.0)


def _tc_step(mi, g, msg, wh):
    return pl.pallas_call(
        _k2_body,
        grid=(EH // BLKE,),
        in_specs=[
            pl.BlockSpec((2, BLKE, H), lambda i: (0, i, 0)),
            pl.BlockSpec((2, BLKE, H), lambda i: (0, i, 0)),
            pl.BlockSpec((2, BLKE, H), lambda i: (0, i, 0)),
            pl.BlockSpec((H, H), lambda i: (0, 0)),
        ],
        out_specs=pl.BlockSpec((2, BLKE, H), lambda i: (0, i, 0)),
        out_shape=jax.ShapeDtypeStruct((2, EH, H), _f32),
    )(mi, g, msg, wh)


def _k3_body(x_b, m_b, gid_b, wox, wom, bo, out, acc, cnt):
    i = pl.program_id(0)
    nsteps = pl.num_programs(0)
    h = jnp.maximum(
        _dotT(x_b[...], wox[...]) + _dotT(m_b[...], wom[...]) + bo[...], 0.0)
    ids = gid_b[...]
    cols = lax.broadcasted_iota(jnp.int32, (1, GP), 1)
    oh = (ids == cols).astype(_f32)
    contrib = lax.dot_general(oh, h, (((0,), (0,)), ((), ())),
                              preferred_element_type=_f32)
    ones = jnp.ones((BLKN, 1), _f32)
    ccol = lax.dot_general(oh, ones, (((0,), (0,)), ((), ())),
                           preferred_element_type=_f32)

    @pl.when(i == 0)
    def _():
        acc[...] = contrib
        cnt[...] = ccol

    @pl.when(i > 0)
    def _():
        acc[...] += contrib
        cnt[...] += ccol

    @pl.when(i == nsteps - 1)
    def _():
        out[...] = acc[...] / jnp.maximum(cnt[...], 1.0)


def _tc_readout(x, m, gid, wox, wom, bo):
    return pl.pallas_call(
        _k3_body,
        grid=(N // BLKN,),
        in_specs=[
            pl.BlockSpec((BLKN, A), lambda i: (i, 0)),
            pl.BlockSpec((BLKN, H), lambda i: (i, 0)),
            pl.BlockSpec((BLKN, 1), lambda i: (i, 0)),
            pl.BlockSpec((H, A), lambda i: (0, 0)),
            pl.BlockSpec((H, H), lambda i: (0, 0)),
            pl.BlockSpec((1, H), lambda i: (0, 0)),
        ],
        out_specs=pl.BlockSpec((GP, H), lambda i: (0, 0)),
        out_shape=jax.ShapeDtypeStruct((GP, H), _f32),
        scratch_shapes=[
            pltpu.VMEM((GP, H), _f32),
            pltpu.VMEM((GP, 1), _f32),
        ],
    )(x, m, gid, wox, wom, bo)


def kernel(x, edge_attr, tree_alpha, W_i, W_h, W_o, b_o, edge_index,
           tree_tgt_nodes, graph_ids):
    src = edge_index[0].astype(jnp.int32)
    dst = edge_index[1].astype(jnp.int32)
    src2 = jnp.concatenate([src[0::2], src[1::2]])
    dst2 = jnp.concatenate([dst[0::2], dst[1::2]])
    ea2 = jnp.stack([edge_attr[0::2], edge_attr[1::2]])
    tt = tree_tgt_nodes.astype(jnp.int32)
    gid = graph_ids.astype(jnp.int32).reshape(N, 1)
    wix = W_i[:, :A]
    wie = W_i[:, A:]
    wox = W_o[:, :A]
    wom = W_o[:, A:]
    bo = b_o.reshape(1, H)

    zero_init = jnp.zeros((N, H), _f32)
    node_alpha = _scatter_T(tree_alpha, tt, zero_init)
    xw = _tc_xw(x, wix)
    g0 = _gather_E(xw, src2).reshape(2, EH, H)
    mi, msg = _tc_init(g0, ea2, wie)
    for _ in range(DEPTH - 1):
        g = _fused_step(msg.reshape(E, H), dst2, src2,
                        node_alpha).reshape(2, EH, H)
        msg = _tc_step(mi, g, msg, W_h)
    m = _scatter_E(msg.reshape(E, H), dst2, node_alpha)
    gr = _tc_readout(x, m, gid, wox, wom, bo)
    return gr[:G]


# speedup vs baseline: 2.6089x; 1.0713x over previous
import functools

import jax
import jax.numpy as jnp
from jax import lax
from jax.experimental import pallas as pl
from jax.experimental.pallas import tpu as pltpu
from jax.experimental.pallas import tpu_sc as plsc

N = 10000
E = 160000
EH = E // 2
A = 128
B = 16
H = 512
DEPTH = 6
T = 40000
G = 100
GP = 104

NC = 2
NS = 16
COLS = 128
NCOL = H // COLS

_mesh = plsc.VectorSubcoreMesh(core_axis_name="c", subcore_axis_name="s")

_f32 = jnp.float32


def _dotT(a, w):
    return lax.dot_general(a, w, (((1,), (1,)), ((), ())),
                           preferred_element_type=_f32)


def _make_scatter(M, CH):
    n_chunks = M // CH
    RCH = 80
    n_rchunks = N // RCH

    @functools.partial(
        pl.kernel,
        out_type=jax.ShapeDtypeStruct((N, H), _f32),
        mesh=_mesh,
        scratch_types=[
            pltpu.VMEM((CH,), jnp.int32),
            pltpu.VMEM((CH, COLS), _f32),
            pltpu.VMEM_SHARED((N, COLS), _f32),
        ],
    )
    def scat(data, idx, init, out, idx_v, data_v, table):
        c = lax.axis_index("c")
        s = lax.axis_index("s")
        for j in range(NCOL // NC):
            col = (c + NC * j) * COLS
            @pl.loop(s, n_rchunks, step=NS)
            def _(r):
                r0 = r * RCH
                pltpu.sync_copy(
                    init.at[pl.ds(r0, RCH), pl.ds(col, COLS)],
                    table.at[pl.ds(r0, RCH)])

            plsc.subcore_barrier()

            @pl.loop(s, n_chunks, step=NS)
            def _(k):
                e0 = k * CH
                pltpu.sync_copy(idx.at[pl.ds(e0, CH)], idx_v)
                pltpu.sync_copy(data.at[pl.ds(e0, CH), pl.ds(col, COLS)],
                                data_v)
                pltpu.sync_copy(data_v, table.at[idx_v], add=True)

            plsc.subcore_barrier()

            @pl.loop(s, n_rchunks, step=NS)
            def _(r):
                r0 = r * RCH
                pltpu.sync_copy(
                    table.at[pl.ds(r0, RCH)],
                    out.at[pl.ds(r0, RCH), pl.ds(col, COLS)])

            plsc.subcore_barrier()

    return scat


def _make_gather(M, W=H, dtype=_f32, CH=128):
    n_chunks = M // CH

    @functools.partial(
        pl.kernel,
        out_type=jax.ShapeDtypeStruct((M, W), dtype),
        mesh=_mesh,
        scratch_types=[
            pltpu.VMEM((CH,), jnp.int32),
            pltpu.VMEM((CH, W), dtype),
            pltpu.SemaphoreType.DMA,
        ],
    )
    def gat(table, idx, out, idx_v, rows_v, sem):
        c = lax.axis_index("c")
        s = lax.axis_index("s")
        w = s * NC + c

        @pl.loop(w, n_chunks, step=NC * NS)
        def _(k):
            e0 = k * CH
            pltpu.sync_copy(idx.at[pl.ds(e0, CH)], idx_v)
            pltpu.async_copy(table.at[idx_v], rows_v, sem).wait()
            pltpu.sync_copy(rows_v, out.at[pl.ds(e0, CH)])

    return gat


_bf16 = jnp.bfloat16
_scatter_E = _make_scatter(E, 128)
_scatter_T = _make_scatter(T, 64)
_gather_E = _make_gather(E, H, _f32)


BCH = 128
NSUB = BCH // 128
NCHF = E // BCH
NTF = 80


@functools.partial(
    pl.kernel,
    out_type=jax.ShapeDtypeStruct((E, H), _f32),
    mesh=_mesh,
    scratch_types=[
        pltpu.VMEM((128,), jnp.int32),
        pltpu.VMEM((128,), jnp.int32),
        pltpu.VMEM((BCH, COLS), _f32),
        pltpu.VMEM((BCH, COLS), _f32),
        pltpu.SemaphoreType.DMA,
        pltpu.SemaphoreType.DMA,
        pltpu.SemaphoreType.DMA,
        pltpu.SemaphoreType.DMA,
        pltpu.VMEM_SHARED((N, COLS), _f32),
    ],
)
def _fused_step(msg, dsti, srci, init, gout,
                ia0, ib0, dat0, dat1, ds0, ds1, ss0, ss1, table):
    ia1 = ia0
    ib1 = ib0
    c = lax.axis_index("c")
    s = lax.axis_index("s")
    idxs = ((ia0, ia1), (ib0, ib1))
    dats = (dat0, dat1)
    dsems = (ds0, ds1)
    ssems = (ss0, ss1)

    for j in range(NCOL // NC):
        col = (c + NC * j) * COLS

        @pl.loop(s, N // 80, step=NS)
        def _(r):
            pltpu.sync_copy(init.at[pl.ds(r * 80, 80), pl.ds(col, COLS)],
                            table.at[pl.ds(r * 80, 80)])

        plsc.subcore_barrier()

        def drain_a(par):
            for r in range(NSUB):
                pltpu.make_async_copy(
                    dats[par].at[pl.ds(r * 128, 128)],
                    table.at[idxs[par][r]], ssems[par]).wait()

        def slot_a(t, par):
            b = s + t * NS

            @pl.when(jnp.logical_and(t >= 2, (b - 2 * NS) < NCHF))
            def _():
                drain_a(par)

            @pl.when(b < NCHF)
            def _():
                cp = pltpu.async_copy(
                    msg.at[pl.ds(b * BCH, BCH), pl.ds(col, COLS)],
                    dats[par], dsems[par])
                for r in range(NSUB):
                    pltpu.sync_copy(
                        dsti.at[pl.ds(b * BCH + r * 128, 128)], idxs[par][r])
                cp.wait()
                for r in range(NSUB):
                    pltpu.async_copy(
                        dats[par].at[pl.ds(r * 128, 128)],
                        table.at[idxs[par][r]], ssems[par], add=True)

        @pl.loop(0, NTF // 2)
        def _(tp):
            slot_a(2 * tp, 0)
            slot_a(2 * tp + 1, 1)

        for t_e, par in ((NTF - 2, 0), (NTF - 1, 1)):
            @pl.when((s + t_e * NS) < NCHF)
            def _():
                drain_a(par)

        plsc.subcore_barrier()

        def slot_b(t, par):
            b = s + t * NS

            @pl.when(jnp.logical_and(t >= 2, (b - 2 * NS) < NCHF))
            def _():
                pltpu.make_async_copy(
                    dats[par],
                    gout.at[pl.ds((b - 2 * NS) * BCH, BCH), pl.ds(col, COLS)],
                    ssems[par]).wait()

            @pl.when(b < NCHF)
            def _():
                for r in range(NSUB):
                    pltpu.sync_copy(
                        srci.at[pl.ds(b * BCH + r * 128, 128)], idxs[par][r])
                for r in range(NSUB):
                    pltpu.async_copy(
                        table.at[idxs[par][r]],
                        dats[par].at[pl.ds(r * 128, 128)], dsems[par])
                for r in range(NSUB):
                    pltpu.make_async_copy(
                        table.at[idxs[par][r]],
                        dats[par].at[pl.ds(r * 128, 128)], dsems[par]).wait()
                pltpu.async_copy(
                    dats[par],
                    gout.at[pl.ds(b * BCH, BCH), pl.ds(col, COLS)],
                    ssems[par])

        @pl.loop(0, NTF // 2)
        def _(tp):
            slot_b(2 * tp, 0)
            slot_b(2 * tp + 1, 1)

        for t_e, par in ((NTF - 2, 0), (NTF - 1, 1)):
            b_e = s + t_e * NS

            @pl.when(b_e < NCHF)
            def _():
                pltpu.make_async_copy(
                    dats[par],
                    gout.at[pl.ds(b_e * BCH, BCH), pl.ds(col, COLS)],
                    ssems[par]).wait()

        plsc.subcore_barrier()


BLKE = 800
BLKN = 1000


def _k0_body(x_b, wix, out):
    out[...] = _dotT(x_b[...], wix[...])


def _tc_xw(x, wix):
    return pl.pallas_call(
        _k0_body,
        grid=(N // BLKN,),
        in_specs=[
            pl.BlockSpec((BLKN, A), lambda i: (i, 0)),
            pl.BlockSpec((H, A), lambda i: (0, 0)),
        ],
        out_specs=pl.BlockSpec((BLKN, H), lambda i: (i, 0)),
        out_shape=jax.ShapeDtypeStruct((N, H), _f32),
    )(x, wix)


def _k1_body(g0, ea, wie, mi_o, msg_o):
    for d in range(2):
        v = g0[d] + _dotT(ea[:, d * B:(d + 1) * B], wie[...])
        mi_o[d] = v.astype(_bf16)
        msg_o[d] = jnp.maximum(v, 0.0)


def _tc_init(g0, ea, wie):
    return pl.pallas_call(
        _k1_body,
        grid=(EH // BLKE,),
        in_specs=[
            pl.BlockSpec((2, BLKE, H), lambda i: (0, i, 0)),
            pl.BlockSpec((BLKE, 2 * B), lambda i: (i, 0)),
            pl.BlockSpec((H, B), lambda i: (0, 0)),
        ],
        out_specs=[
            pl.BlockSpec((2, BLKE, H), lambda i: (0, i, 0)),
            pl.BlockSpec((2, BLKE, H), lambda i: (0, i, 0)),
        ],
        out_shape=[
            jax.ShapeDtypeStruct((2, EH, H), _bf16),
            jax.ShapeDtypeStruct((2, EH, H), _f32),
        ],
    )(g0, ea, wie)


def _k2_body(mi, g, msg, wh, out):
    for d in range(2):
        acc = g[d] - msg[1 - d]
        v = jnp.maximum(mi[d].astype(_f32) + _dotT(acc, wh[...]), 0.0)
        out[d] = v


def _tc_step(mi, g, msg, wh):
    return pl.pallas_call(
        _k2_body,
        grid=(EH // BLKE,),
        in_specs=[
            pl.BlockSpec((2, BLKE, H), lambda i: (0, i, 0)),
            pl.BlockSpec((2, BLKE, H), lambda i: (0, i, 0)),
            pl.BlockSpec((2, BLKE, H), lambda i: (0, i, 0)),
            pl.BlockSpec((H, H), lambda i: (0, 0)),
        ],
        out_specs=pl.BlockSpec((2, BLKE, H), lambda i: (0, i, 0)),
        out_shape=jax.ShapeDtypeStruct((2, EH, H), _f32),
    )(mi, g, msg, wh)


def _k3_body(x_b, m_b, gid_b, wox, wom, bo, out, acc, cnt):
    i = pl.program_id(0)
    nsteps = pl.num_programs(0)
    h = jnp.maximum(
        _dotT(x_b[...], wox[...]) + _dotT(m_b[...], wom[...]) + bo[...], 0.0)
    ids = gid_b[...]
    cols = lax.broadcasted_iota(jnp.int32, (1, GP), 1)
    oh = (ids == cols).astype(_f32)
    contrib = lax.dot_general(oh, h, (((0,), (0,)), ((), ())),
                              preferred_element_type=_f32)
    ones = jnp.ones((BLKN, 1), _f32)
    ccol = lax.dot_general(oh, ones, (((0,), (0,)), ((), ())),
                           preferred_element_type=_f32)

    @pl.when(i == 0)
    def _():
        acc[...] = contrib
        cnt[...] = ccol

    @pl.when(i > 0)
    def _():
        acc[...] += contrib
        cnt[...] += ccol

    @pl.when(i == nsteps - 1)
    def _():
        out[...] = acc[...] / jnp.maximum(cnt[...], 1.0)


def _tc_readout(x, m, gid, wox, wom, bo):
    return pl.pallas_call(
        _k3_body,
        grid=(N // BLKN,),
        in_specs=[
            pl.BlockSpec((BLKN, A), lambda i: (i, 0)),
            pl.BlockSpec((BLKN, H), lambda i: (i, 0)),
            pl.BlockSpec((BLKN, 1), lambda i: (i, 0)),
            pl.BlockSpec((H, A), lambda i: (0, 0)),
            pl.BlockSpec((H, H), lambda i: (0, 0)),
            pl.BlockSpec((1, H), lambda i: (0, 0)),
        ],
        out_specs=pl.BlockSpec((GP, H), lambda i: (0, 0)),
        out_shape=jax.ShapeDtypeStruct((GP, H), _f32),
        scratch_shapes=[
            pltpu.VMEM((GP, H), _f32),
            pltpu.VMEM((GP, 1), _f32),
        ],
    )(x, m, gid, wox, wom, bo)


def kernel(x, edge_attr, tree_alpha, W_i, W_h, W_o, b_o, edge_index,
           tree_tgt_nodes, graph_ids):
    src = edge_index[0].astype(jnp.int32)
    dst = edge_index[1].astype(jnp.int32)
    src2 = jnp.concatenate([src[0::2], src[1::2]])
    dst2 = jnp.concatenate([dst[0::2], dst[1::2]])
    ea2 = edge_attr.reshape(EH, 2 * B)
    tt = tree_tgt_nodes.astype(jnp.int32)
    gid = graph_ids.astype(jnp.int32).reshape(N, 1)
    wix = W_i[:, :A]
    wie = W_i[:, A:]
    wox = W_o[:, :A]
    wom = W_o[:, A:]
    bo = b_o.reshape(1, H)

    zero_init = jnp.zeros((N, H), _f32)
    node_alpha = _scatter_T(tree_alpha, tt, zero_init)
    xw = _tc_xw(x, wix)
    g0 = _gather_E(xw, src2).reshape(2, EH, H)
    mi, msg = _tc_init(g0, ea2, wie)
    for _ in range(DEPTH - 1):
        g = _fused_step(msg.reshape(E, H), dst2, src2,
                        node_alpha).reshape(2, EH, H)
        msg = _tc_step(mi, g, msg, W_h)
    m = _scatter_E(msg.reshape(E, H), dst2, node_alpha)
    gr = _tc_readout(x, m, gid, wox, wom, bo)
    return gr[:G]


# speedup vs baseline: 2.7079x; 1.0380x over previous
import functools

import jax
import jax.numpy as jnp
from jax import lax
from jax.experimental import pallas as pl
from jax.experimental.pallas import tpu as pltpu
from jax.experimental.pallas import tpu_sc as plsc

N = 10000
E = 160000
EH = E // 2
A = 128
B = 16
H = 512
DEPTH = 6
T = 40000
G = 100
GP = 104

NC = 2
NS = 16
COLS = 128
NCOL = H // COLS

_mesh = plsc.VectorSubcoreMesh(core_axis_name="c", subcore_axis_name="s")

_f32 = jnp.float32


def _dotT(a, w):
    return lax.dot_general(a, w, (((1,), (1,)), ((), ())),
                           preferred_element_type=_f32)


def _make_scatter(M, CH):
    n_chunks = M // CH
    nt = 2 * ((n_chunks // NS + 2) // 2)
    RCH = 80
    n_rchunks = N // RCH

    @functools.partial(
        pl.kernel,
        out_type=jax.ShapeDtypeStruct((N, H), _f32),
        mesh=_mesh,
        scratch_types=[
            pltpu.VMEM((CH,), jnp.int32),
            pltpu.VMEM((CH,), jnp.int32),
            pltpu.VMEM((CH, COLS), _f32),
            pltpu.VMEM((CH, COLS), _f32),
            pltpu.SemaphoreType.DMA,
            pltpu.SemaphoreType.DMA,
            pltpu.SemaphoreType.DMA,
            pltpu.SemaphoreType.DMA,
            pltpu.VMEM_SHARED((N, COLS), _f32),
        ],
    )
    def scat(data, idx, init, out, i0, i1, d0, d1, ds0, ds1, ss0, ss1, table):
        c = lax.axis_index("c")
        s = lax.axis_index("s")
        idxs = (i0, i1)
        dats = (d0, d1)
        dsems = (ds0, ds1)
        ssems = (ss0, ss1)
        for j in range(NCOL // NC):
            col = (c + NC * j) * COLS
            @pl.loop(s, n_rchunks, step=NS)
            def _(r):
                r0 = r * RCH
                pltpu.sync_copy(
                    init.at[pl.ds(r0, RCH), pl.ds(col, COLS)],
                    table.at[pl.ds(r0, RCH)])

            plsc.subcore_barrier()

            def slot(t, par):
                b = s + t * NS

                @pl.when(jnp.logical_and(t >= 2, (b - 2 * NS) < n_chunks))
                def _():
                    pltpu.make_async_copy(dats[par], table.at[idxs[par]],
                                          ssems[par]).wait()

                @pl.when(b < n_chunks)
                def _():
                    cp = pltpu.async_copy(
                        data.at[pl.ds(b * CH, CH), pl.ds(col, COLS)],
                        dats[par], dsems[par])
                    pltpu.sync_copy(idx.at[pl.ds(b * CH, CH)], idxs[par])
                    cp.wait()
                    pltpu.async_copy(dats[par], table.at[idxs[par]],
                                     ssems[par], add=True)

            @pl.loop(0, nt // 2)
            def _(tp):
                slot(2 * tp, 0)
                slot(2 * tp + 1, 1)

            for t_e, par in ((nt - 2, 0), (nt - 1, 1)):
                @pl.when((s + t_e * NS) < n_chunks)
                def _():
                    pltpu.make_async_copy(dats[par], table.at[idxs[par]],
                                          ssems[par]).wait()

            plsc.subcore_barrier()

            @pl.loop(s, n_rchunks, step=NS)
            def _(r):
                r0 = r * RCH
                pltpu.sync_copy(
                    table.at[pl.ds(r0, RCH)],
                    out.at[pl.ds(r0, RCH), pl.ds(col, COLS)])

            plsc.subcore_barrier()

    return scat


def _make_gather(M, W=H, dtype=_f32, CH=64):
    n_chunks = M // CH
    nw = NC * NS
    nt = 2 * ((n_chunks // nw + 2) // 2)

    @functools.partial(
        pl.kernel,
        out_type=jax.ShapeDtypeStruct((M, W), dtype),
        mesh=_mesh,
        scratch_types=[
            pltpu.VMEM((CH,), jnp.int32),
            pltpu.VMEM((CH,), jnp.int32),
            pltpu.VMEM((CH, W), dtype),
            pltpu.VMEM((CH, W), dtype),
            pltpu.SemaphoreType.DMA,
            pltpu.SemaphoreType.DMA,
            pltpu.SemaphoreType.DMA,
            pltpu.SemaphoreType.DMA,
        ],
    )
    def gat(table, idx, out, i0, i1, r0, r1, gs0, gs1, ws0, ws1):
        c = lax.axis_index("c")
        s = lax.axis_index("s")
        w = s * NC + c
        idxs = (i0, i1)
        rows = (r0, r1)
        gsems = (gs0, gs1)
        wsems = (ws0, ws1)

        def slot(t, par):
            b = w + t * nw

            @pl.when(jnp.logical_and(t >= 2, (b - 2 * nw) < n_chunks))
            def _():
                pltpu.make_async_copy(
                    rows[par],
                    out.at[pl.ds((b - 2 * nw) * CH, CH)], wsems[par]).wait()

            @pl.when(b < n_chunks)
            def _():
                pltpu.sync_copy(idx.at[pl.ds(b * CH, CH)], idxs[par])
                pltpu.async_copy(table.at[idxs[par]], rows[par],
                                 gsems[par]).wait()
                pltpu.async_copy(rows[par], out.at[pl.ds(b * CH, CH)],
                                 wsems[par])

        @pl.loop(0, nt // 2)
        def _(tp):
            slot(2 * tp, 0)
            slot(2 * tp + 1, 1)

        for t_e, par in ((nt - 2, 0), (nt - 1, 1)):
            b_e = w + t_e * nw

            @pl.when(b_e < n_chunks)
            def _():
                pltpu.make_async_copy(
                    rows[par], out.at[pl.ds(b_e * CH, CH)], wsems[par]).wait()

    return gat


_bf16 = jnp.bfloat16
_scatter_E = _make_scatter(E, 128)
_scatter_T = _make_scatter(T, 64)
_gather_E = _make_gather(E, H, _f32)


BCH = 128
NSUB = BCH // 128
NCHF = E // BCH
NTF = 80


@functools.partial(
    pl.kernel,
    out_type=jax.ShapeDtypeStruct((E, H), _f32),
    mesh=_mesh,
    scratch_types=[
        pltpu.VMEM((128,), jnp.int32),
        pltpu.VMEM((128,), jnp.int32),
        pltpu.VMEM((BCH, COLS), _f32),
        pltpu.VMEM((BCH, COLS), _f32),
        pltpu.SemaphoreType.DMA,
        pltpu.SemaphoreType.DMA,
        pltpu.SemaphoreType.DMA,
        pltpu.SemaphoreType.DMA,
        pltpu.VMEM_SHARED((N, COLS), _f32),
    ],
)
def _fused_step(msg, dsti, srci, init, gout,
                ia0, ib0, dat0, dat1, ds0, ds1, ss0, ss1, table):
    ia1 = ia0
    ib1 = ib0
    c = lax.axis_index("c")
    s = lax.axis_index("s")
    idxs = ((ia0, ia1), (ib0, ib1))
    dats = (dat0, dat1)
    dsems = (ds0, ds1)
    ssems = (ss0, ss1)

    for j in range(NCOL // NC):
        col = (c + NC * j) * COLS

        @pl.loop(s, N // 80, step=NS)
        def _(r):
            pltpu.sync_copy(init.at[pl.ds(r * 80, 80), pl.ds(col, COLS)],
                            table.at[pl.ds(r * 80, 80)])

        plsc.subcore_barrier()

        def drain_a(par):
            for r in range(NSUB):
                pltpu.make_async_copy(
                    dats[par].at[pl.ds(r * 128, 128)],
                    table.at[idxs[par][r]], ssems[par]).wait()

        def slot_a(t, par):
            b = s + t * NS

            @pl.when(jnp.logical_and(t >= 2, (b - 2 * NS) < NCHF))
            def _():
                drain_a(par)

            @pl.when(b < NCHF)
            def _():
                cp = pltpu.async_copy(
                    msg.at[pl.ds(b * BCH, BCH), pl.ds(col, COLS)],
                    dats[par], dsems[par])
                for r in range(NSUB):
                    pltpu.sync_copy(
                        dsti.at[pl.ds(b * BCH + r * 128, 128)], idxs[par][r])
                cp.wait()
                for r in range(NSUB):
                    pltpu.async_copy(
                        dats[par].at[pl.ds(r * 128, 128)],
                        table.at[idxs[par][r]], ssems[par], add=True)

        @pl.loop(0, NTF // 2)
        def _(tp):
            slot_a(2 * tp, 0)
            slot_a(2 * tp + 1, 1)

        for t_e, par in ((NTF - 2, 0), (NTF - 1, 1)):
            @pl.when((s + t_e * NS) < NCHF)
            def _():
                drain_a(par)

        plsc.subcore_barrier()

        def slot_b(t, par):
            b = s + t * NS

            @pl.when(jnp.logical_and(t >= 2, (b - 2 * NS) < NCHF))
            def _():
                pltpu.make_async_copy(
                    dats[par],
                    gout.at[pl.ds((b - 2 * NS) * BCH, BCH), pl.ds(col, COLS)],
                    ssems[par]).wait()

            @pl.when(b < NCHF)
            def _():
                for r in range(NSUB):
                    pltpu.sync_copy(
                        srci.at[pl.ds(b * BCH + r * 128, 128)], idxs[par][r])
                for r in range(NSUB):
                    pltpu.async_copy(
                        table.at[idxs[par][r]],
                        dats[par].at[pl.ds(r * 128, 128)], dsems[par])
                for r in range(NSUB):
                    pltpu.make_async_copy(
                        table.at[idxs[par][r]],
                        dats[par].at[pl.ds(r * 128, 128)], dsems[par]).wait()
                pltpu.async_copy(
                    dats[par],
                    gout.at[pl.ds(b * BCH, BCH), pl.ds(col, COLS)],
                    ssems[par])

        @pl.loop(0, NTF // 2)
        def _(tp):
            slot_b(2 * tp, 0)
            slot_b(2 * tp + 1, 1)

        for t_e, par in ((NTF - 2, 0), (NTF - 1, 1)):
            b_e = s + t_e * NS

            @pl.when(b_e < NCHF)
            def _():
                pltpu.make_async_copy(
                    dats[par],
                    gout.at[pl.ds(b_e * BCH, BCH), pl.ds(col, COLS)],
                    ssems[par]).wait()

        plsc.subcore_barrier()


BLKE = 800
BLKN = 1000


def _k0_body(x_b, wix, out):
    out[...] = _dotT(x_b[...], wix[...])


def _tc_xw(x, wix):
    return pl.pallas_call(
        _k0_body,
        grid=(N // BLKN,),
        in_specs=[
            pl.BlockSpec((BLKN, A), lambda i: (i, 0)),
            pl.BlockSpec((H, A), lambda i: (0, 0)),
        ],
        out_specs=pl.BlockSpec((BLKN, H), lambda i: (i, 0)),
        out_shape=jax.ShapeDtypeStruct((N, H), _f32),
    )(x, wix)


def _k1_body(g0, ea, wie, mi_o, msg_o):
    for d in range(2):
        v = g0[d] + _dotT(ea[:, d * B:(d + 1) * B], wie[...])
        mi_o[d] = v.astype(_bf16)
        msg_o[d] = jnp.maximum(v, 0.0)


def _tc_init(g0, ea, wie):
    return pl.pallas_call(
        _k1_body,
        grid=(EH // BLKE,),
        in_specs=[
            pl.BlockSpec((2, BLKE, H), lambda i: (0, i, 0)),
            pl.BlockSpec((BLKE, 2 * B), lambda i: (i, 0)),
            pl.BlockSpec((H, B), lambda i: (0, 0)),
        ],
        out_specs=[
            pl.BlockSpec((2, BLKE, H), lambda i: (0, i, 0)),
            pl.BlockSpec((2, BLKE, H), lambda i: (0, i, 0)),
        ],
        out_shape=[
            jax.ShapeDtypeStruct((2, EH, H), _bf16),
            jax.ShapeDtypeStruct((2, EH, H), _f32),
        ],
    )(g0, ea, wie)


def _k2_body(mi, g, msg, wh, out):
    for d in range(2):
        acc = g[d] - msg[1 - d]
        v = jnp.maximum(mi[d].astype(_f32) + _dotT(acc, wh[...]), 0.0)
        out[d] = v


def _tc_step(mi, g, msg, wh):
    return pl.pallas_call(
        _k2_body,
        grid=(EH // BLKE,),
        in_specs=[
            pl.BlockSpec((2, BLKE, H), lambda i: (0, i, 0)),
            pl.BlockSpec((2, BLKE, H), lambda i: (0, i, 0)),
            pl.BlockSpec((2, BLKE, H), lambda i: (0, i, 0)),
            pl.BlockSpec((H, H), lambda i: (0, 0)),
        ],
        out_specs=pl.BlockSpec((2, BLKE, H), lambda i: (0, i, 0)),
        out_shape=jax.ShapeDtypeStruct((2, EH, H), _f32),
    )(mi, g, msg, wh)


def _k3_body(x_b, m_b, gid_b, wox, wom, bo, out, acc, cnt):
    i = pl.program_id(0)
    nsteps = pl.num_programs(0)
    h = jnp.maximum(
        _dotT(x_b[...], wox[...]) + _dotT(m_b[...], wom[...]) + bo[...], 0.0)
    ids = gid_b[...]
    cols = lax.broadcasted_iota(jnp.int32, (1, GP), 1)
    oh = (ids == cols).astype(_f32)
    contrib = lax.dot_general(oh, h, (((0,), (0,)), ((), ())),
                              preferred_element_type=_f32)
    ones = jnp.ones((BLKN, 1), _f32)
    ccol = lax.dot_general(oh, ones, (((0,), (0,)), ((), ())),
                           preferred_element_type=_f32)

    @pl.when(i == 0)
    def _():
        acc[...] = contrib
        cnt[...] = ccol

    @pl.when(i > 0)
    def _():
        acc[...] += contrib
        cnt[...] += ccol

    @pl.when(i == nsteps - 1)
    def _():
        out[...] = acc[...] / jnp.maximum(cnt[...], 1.0)


def _tc_readout(x, m, gid, wox, wom, bo):
    return pl.pallas_call(
        _k3_body,
        grid=(N // BLKN,),
        in_specs=[
            pl.BlockSpec((BLKN, A), lambda i: (i, 0)),
            pl.BlockSpec((BLKN, H), lambda i: (i, 0)),
            pl.BlockSpec((BLKN, 1), lambda i: (i, 0)),
            pl.BlockSpec((H, A), lambda i: (0, 0)),
            pl.BlockSpec((H, H), lambda i: (0, 0)),
            pl.BlockSpec((1, H), lambda i: (0, 0)),
        ],
        out_specs=pl.BlockSpec((GP, H), lambda i: (0, 0)),
        out_shape=jax.ShapeDtypeStruct((GP, H), _f32),
        scratch_shapes=[
            pltpu.VMEM((GP, H), _f32),
            pltpu.VMEM((GP, 1), _f32),
        ],
    )(x, m, gid, wox, wom, bo)


def kernel(x, edge_attr, tree_alpha, W_i, W_h, W_o, b_o, edge_index,
           tree_tgt_nodes, graph_ids):
    src = edge_index[0].astype(jnp.int32)
    dst = edge_index[1].astype(jnp.int32)
    src2 = jnp.concatenate([src[0::2], src[1::2]])
    dst2 = jnp.concatenate([dst[0::2], dst[1::2]])
    ea2 = edge_attr.reshape(EH, 2 * B)
    tt = tree_tgt_nodes.astype(jnp.int32)
    gid = graph_ids.astype(jnp.int32).reshape(N, 1)
    wix = W_i[:, :A]
    wie = W_i[:, A:]
    wox = W_o[:, :A]
    wom = W_o[:, A:]
    bo = b_o.reshape(1, H)

    zero_init = jnp.zeros((N, H), _f32)
    node_alpha = _scatter_T(tree_alpha, tt, zero_init)
    xw = _tc_xw(x, wix)
    g0 = _gather_E(xw, src2).reshape(2, EH, H)
    mi, msg = _tc_init(g0, ea2, wie)
    for _ in range(DEPTH - 1):
        g = _fused_step(msg.reshape(E, H), dst2, src2,
                        node_alpha).reshape(2, EH, H)
        msg = _tc_step(mi, g, msg, W_h)
    m = _scatter_E(msg.reshape(E, H), dst2, node_alpha)
    gr = _tc_readout(x, m, gid, wox, wom, bo)
    return gr[:G]


# speedup vs baseline: 2.7143x; 1.0024x over previous
import functools

import jax
import jax.numpy as jnp
from jax import lax
from jax.experimental import pallas as pl
from jax.experimental.pallas import tpu as pltpu
from jax.experimental.pallas import tpu_sc as plsc

N = 10000
E = 160000
EH = E // 2
A = 128
B = 16
H = 512
DEPTH = 6
T = 40000
G = 100
GP = 104

NC = 2
NS = 16
COLS = 128
NCOL = H // COLS

_mesh = plsc.VectorSubcoreMesh(core_axis_name="c", subcore_axis_name="s")

_f32 = jnp.float32


def _dotT(a, w):
    return lax.dot_general(a, w, (((1,), (1,)), ((), ())),
                           preferred_element_type=_f32)


def _make_scatter(M, CH):
    n_chunks = M // CH
    nt = 2 * ((n_chunks // NS + 2) // 2)
    RCH = 80
    n_rchunks = N // RCH

    @functools.partial(
        pl.kernel,
        out_type=jax.ShapeDtypeStruct((N, H), _f32),
        mesh=_mesh,
        scratch_types=[
            pltpu.VMEM((CH,), jnp.int32),
            pltpu.VMEM((CH,), jnp.int32),
            pltpu.VMEM((CH, COLS), _f32),
            pltpu.VMEM((CH, COLS), _f32),
            pltpu.SemaphoreType.DMA,
            pltpu.SemaphoreType.DMA,
            pltpu.SemaphoreType.DMA,
            pltpu.SemaphoreType.DMA,
            pltpu.VMEM_SHARED((N, COLS), _f32),
        ],
    )
    def scat(data, idx, init, out, i0, i1, d0, d1, ds0, ds1, ss0, ss1, table):
        c = lax.axis_index("c")
        s = lax.axis_index("s")
        idxs = (i0, i1)
        dats = (d0, d1)
        dsems = (ds0, ds1)
        ssems = (ss0, ss1)
        for j in range(NCOL // NC):
            col = (c + NC * j) * COLS
            @pl.loop(s, n_rchunks, step=NS)
            def _(r):
                r0 = r * RCH
                pltpu.sync_copy(
                    init.at[pl.ds(r0, RCH), pl.ds(col, COLS)],
                    table.at[pl.ds(r0, RCH)])

            plsc.subcore_barrier()

            def slot(t, par):
                b = s + t * NS

                @pl.when(jnp.logical_and(t >= 2, (b - 2 * NS) < n_chunks))
                def _():
                    pltpu.make_async_copy(dats[par], table.at[idxs[par]],
                                          ssems[par]).wait()

                @pl.when(b < n_chunks)
                def _():
                    cp = pltpu.async_copy(
                        data.at[pl.ds(b * CH, CH), pl.ds(col, COLS)],
                        dats[par], dsems[par])
                    pltpu.sync_copy(idx.at[pl.ds(b * CH, CH)], idxs[par])
                    cp.wait()
                    pltpu.async_copy(dats[par], table.at[idxs[par]],
                                     ssems[par], add=True)

            @pl.loop(0, nt // 2)
            def _(tp):
                slot(2 * tp, 0)
                slot(2 * tp + 1, 1)

            for t_e, par in ((nt - 2, 0), (nt - 1, 1)):
                @pl.when((s + t_e * NS) < n_chunks)
                def _():
                    pltpu.make_async_copy(dats[par], table.at[idxs[par]],
                                          ssems[par]).wait()

            plsc.subcore_barrier()

            @pl.loop(s, n_rchunks, step=NS)
            def _(r):
                r0 = r * RCH
                pltpu.sync_copy(
                    table.at[pl.ds(r0, RCH)],
                    out.at[pl.ds(r0, RCH), pl.ds(col, COLS)])

            plsc.subcore_barrier()

    return scat


def _make_gather(M, W=H, dtype=_f32, CH=64):
    n_chunks = M // CH
    nw = NC * NS
    nt = 2 * ((n_chunks // nw + 2) // 2)

    @functools.partial(
        pl.kernel,
        out_type=jax.ShapeDtypeStruct((M, W), dtype),
        mesh=_mesh,
        scratch_types=[
            pltpu.VMEM((CH,), jnp.int32),
            pltpu.VMEM((CH,), jnp.int32),
            pltpu.VMEM((CH, W), dtype),
            pltpu.VMEM((CH, W), dtype),
            pltpu.SemaphoreType.DMA,
            pltpu.SemaphoreType.DMA,
            pltpu.SemaphoreType.DMA,
            pltpu.SemaphoreType.DMA,
        ],
    )
    def gat(table, idx, out, i0, i1, r0, r1, gs0, gs1, ws0, ws1):
        c = lax.axis_index("c")
        s = lax.axis_index("s")
        w = s * NC + c
        idxs = (i0, i1)
        rows = (r0, r1)
        gsems = (gs0, gs1)
        wsems = (ws0, ws1)

        def slot(t, par):
            b = w + t * nw

            @pl.when(jnp.logical_and(t >= 2, (b - 2 * nw) < n_chunks))
            def _():
                pltpu.make_async_copy(
                    rows[par],
                    out.at[pl.ds((b - 2 * nw) * CH, CH)], wsems[par]).wait()

            @pl.when(b < n_chunks)
            def _():
                pltpu.sync_copy(idx.at[pl.ds(b * CH, CH)], idxs[par])
                pltpu.async_copy(table.at[idxs[par]], rows[par],
                                 gsems[par]).wait()
                pltpu.async_copy(rows[par], out.at[pl.ds(b * CH, CH)],
                                 wsems[par])

        @pl.loop(0, nt // 2)
        def _(tp):
            slot(2 * tp, 0)
            slot(2 * tp + 1, 1)

        for t_e, par in ((nt - 2, 0), (nt - 1, 1)):
            b_e = w + t_e * nw

            @pl.when(b_e < n_chunks)
            def _():
                pltpu.make_async_copy(
                    rows[par], out.at[pl.ds(b_e * CH, CH)], wsems[par]).wait()

    return gat


_bf16 = jnp.bfloat16
_scatter_h = _make_scatter(EH, 128)
_scatter_T = _make_scatter(T, 64)
_gather_h = _make_gather(EH, H, _f32)


BCH = 128


def _make_fused(M, dump):
    nchf = M // BCH
    ntf = 2 * ((nchf // NS + 2) // 2)

    out_types = [jax.ShapeDtypeStruct((M, H), _f32)]
    if dump:
        out_types.append(jax.ShapeDtypeStruct((N, H), _f32))

    @functools.partial(
        pl.kernel,
        out_type=tuple(out_types),
        mesh=_mesh,
        scratch_types=[
            pltpu.VMEM((BCH,), jnp.int32),
            pltpu.VMEM((BCH,), jnp.int32),
            pltpu.VMEM((BCH, COLS), _f32),
            pltpu.VMEM((BCH, COLS), _f32),
            pltpu.SemaphoreType.DMA,
            pltpu.SemaphoreType.DMA,
            pltpu.SemaphoreType.DMA,
            pltpu.SemaphoreType.DMA,
            pltpu.VMEM_SHARED((N, COLS), _f32),
        ],
    )
    def fused(msg, dsti, srci, init, *outs_and_scratch):
        if dump:
            gout, tout = outs_and_scratch[0], outs_and_scratch[1]
            rest = outs_and_scratch[2:]
        else:
            gout = outs_and_scratch[0]
            rest = outs_and_scratch[1:]
        ia0, ib0, dat0, dat1, ds0, ds1, ss0, ss1, table = rest
        c = lax.axis_index("c")
        s = lax.axis_index("s")
        idxs = (ia0, ib0)
        dats = (dat0, dat1)
        dsems = (ds0, ds1)
        ssems = (ss0, ss1)

        for j in range(NCOL // NC):
            col = (c + NC * j) * COLS

            @pl.loop(s, N // 80, step=NS)
            def _(r):
                pltpu.sync_copy(init.at[pl.ds(r * 80, 80), pl.ds(col, COLS)],
                                table.at[pl.ds(r * 80, 80)])

            plsc.subcore_barrier()

            def slot_a(t, par):
                b = s + t * NS

                @pl.when(jnp.logical_and(t >= 2, (b - 2 * NS) < nchf))
                def _():
                    pltpu.make_async_copy(dats[par], table.at[idxs[par]],
                                          ssems[par]).wait()

                @pl.when(b < nchf)
                def _():
                    cp = pltpu.async_copy(
                        msg.at[pl.ds(b * BCH, BCH), pl.ds(col, COLS)],
                        dats[par], dsems[par])
                    pltpu.sync_copy(dsti.at[pl.ds(b * BCH, BCH)], idxs[par])
                    cp.wait()
                    pltpu.async_copy(dats[par], table.at[idxs[par]],
                                     ssems[par], add=True)

            @pl.loop(0, ntf // 2)
            def _(tp):
                slot_a(2 * tp, 0)
                slot_a(2 * tp + 1, 1)

            for t_e, par in ((ntf - 2, 0), (ntf - 1, 1)):
                @pl.when((s + t_e * NS) < nchf)
                def _():
                    pltpu.make_async_copy(dats[par], table.at[idxs[par]],
                                          ssems[par]).wait()

            plsc.subcore_barrier()

            if dump:
                @pl.loop(s, N // 80, step=NS)
                def _(r):
                    pltpu.sync_copy(
                        table.at[pl.ds(r * 80, 80)],
                        tout.at[pl.ds(r * 80, 80), pl.ds(col, COLS)])

            def slot_b(t, par):
                b = s + t * NS

                @pl.when(jnp.logical_and(t >= 2, (b - 2 * NS) < nchf))
                def _():
                    pltpu.make_async_copy(
                        dats[par],
                        gout.at[pl.ds((b - 2 * NS) * BCH, BCH),
                                pl.ds(col, COLS)],
                        ssems[par]).wait()

                @pl.when(b < nchf)
                def _():
                    pltpu.sync_copy(srci.at[pl.ds(b * BCH, BCH)], idxs[par])
                    pltpu.async_copy(table.at[idxs[par]], dats[par],
                                     dsems[par]).wait()
                    pltpu.async_copy(
                        dats[par],
                        gout.at[pl.ds(b * BCH, BCH), pl.ds(col, COLS)],
                        ssems[par])

            @pl.loop(0, ntf // 2)
            def _(tp):
                slot_b(2 * tp, 0)
                slot_b(2 * tp + 1, 1)

            for t_e, par in ((ntf - 2, 0), (ntf - 1, 1)):
                b_e = s + t_e * NS

                @pl.when(b_e < nchf)
                def _():
                    pltpu.make_async_copy(
                        dats[par],
                        gout.at[pl.ds(b_e * BCH, BCH), pl.ds(col, COLS)],
                        ssems[par]).wait()

            plsc.subcore_barrier()

    return fused


_fused_half = _make_fused(EH, dump=True)


BLKE = 800
BLKN = 1000


def _k0_body(x_b, wix, out):
    out[...] = _dotT(x_b[...], wix[...])


def _tc_xw(x, wix):
    return pl.pallas_call(
        _k0_body,
        grid=(N // BLKN,),
        in_specs=[
            pl.BlockSpec((BLKN, A), lambda i: (i, 0)),
            pl.BlockSpec((H, A), lambda i: (0, 0)),
        ],
        out_specs=pl.BlockSpec((BLKN, H), lambda i: (i, 0)),
        out_shape=jax.ShapeDtypeStruct((N, H), _f32),
    )(x, wix)


def _k1_body(g0, ea, wie, mi_o, msg_o):
    for d in range(2):
        v = g0[d] + _dotT(ea[:, d * B:(d + 1) * B], wie[...])
        mi_o[d] = v.astype(_bf16)
        msg_o[d] = jnp.maximum(v, 0.0)


def _make_tc_init(rows):
    def init(g0, ea, wie):
        return pl.pallas_call(
            _k1_body,
            grid=(rows // BLKE,),
            in_specs=[
                pl.BlockSpec((2, BLKE, H), lambda i: (0, i, 0)),
                pl.BlockSpec((BLKE, 2 * B), lambda i: (i, 0)),
                pl.BlockSpec((H, B), lambda i: (0, 0)),
            ],
            out_specs=[
                pl.BlockSpec((2, BLKE, H), lambda i: (0, i, 0)),
                pl.BlockSpec((2, BLKE, H), lambda i: (0, i, 0)),
            ],
            out_shape=[
                jax.ShapeDtypeStruct((2, rows, H), _bf16),
                jax.ShapeDtypeStruct((2, rows, H), _f32),
            ],
        )(g0, ea, wie)

    return init


def _k2_body(mi, g, msg, wh, out):
    for d in range(2):
        acc = g[d] - msg[1 - d]
        v = jnp.maximum(mi[d].astype(_f32) + _dotT(acc, wh[...]), 0.0)
        out[d] = v


def _make_tc_step(rows):
    def step(mi, g, msg, wh):
        return pl.pallas_call(
            _k2_body,
            grid=(rows // BLKE,),
            in_specs=[
                pl.BlockSpec((2, BLKE, H), lambda i: (0, i, 0)),
                pl.BlockSpec((2, BLKE, H), lambda i: (0, i, 0)),
                pl.BlockSpec((2, BLKE, H), lambda i: (0, i, 0)),
                pl.BlockSpec((H, H), lambda i: (0, 0)),
            ],
            out_specs=pl.BlockSpec((2, BLKE, H), lambda i: (0, i, 0)),
            out_shape=jax.ShapeDtypeStruct((2, rows, H), _f32),
        )(mi, g, msg, wh)

    return step


EH2 = EH // 2
_tc_init_h = _make_tc_init(EH2)
_tc_step_h = _make_tc_step(EH2)


def _k3_body(x_b, m_b, gid_b, wox, wom, bo, out, acc, cnt):
    i = pl.program_id(0)
    nsteps = pl.num_programs(0)
    h = jnp.maximum(
        _dotT(x_b[...], wox[...]) + _dotT(m_b[...], wom[...]) + bo[...], 0.0)
    ids = gid_b[...]
    cols = lax.broadcasted_iota(jnp.int32, (1, GP), 1)
    oh = (ids == cols).astype(_f32)
    contrib = lax.dot_general(oh, h, (((0,), (0,)), ((), ())),
                              preferred_element_type=_f32)
    ones = jnp.ones((BLKN, 1), _f32)
    ccol = lax.dot_general(oh, ones, (((0,), (0,)), ((), ())),
                           preferred_element_type=_f32)

    @pl.when(i == 0)
    def _():
        acc[...] = contrib
        cnt[...] = ccol

    @pl.when(i > 0)
    def _():
        acc[...] += contrib
        cnt[...] += ccol

    @pl.when(i == nsteps - 1)
    def _():
        out[...] = acc[...] / jnp.maximum(cnt[...], 1.0)


def _tc_readout(x, m, gid, wox, wom, bo):
    return pl.pallas_call(
        _k3_body,
        grid=(N // BLKN,),
        in_specs=[
            pl.BlockSpec((BLKN, A), lambda i: (i, 0)),
            pl.BlockSpec((BLKN, H), lambda i: (i, 0)),
            pl.BlockSpec((BLKN, 1), lambda i: (i, 0)),
            pl.BlockSpec((H, A), lambda i: (0, 0)),
            pl.BlockSpec((H, H), lambda i: (0, 0)),
            pl.BlockSpec((1, H), lambda i: (0, 0)),
        ],
        out_specs=pl.BlockSpec((GP, H), lambda i: (0, 0)),
        out_shape=jax.ShapeDtypeStruct((GP, H), _f32),
        scratch_shapes=[
            pltpu.VMEM((GP, H), _f32),
            pltpu.VMEM((GP, 1), _f32),
        ],
    )(x, m, gid, wox, wom, bo)


def kernel(x, edge_attr, tree_alpha, W_i, W_h, W_o, b_o, edge_index,
           tree_tgt_nodes, graph_ids):
    src = edge_index[0].astype(jnp.int32)
    dst = edge_index[1].astype(jnp.int32)
    src2 = jnp.concatenate([src[0::2], src[1::2]])
    dst2 = jnp.concatenate([dst[0::2], dst[1::2]])
    src_a = jnp.concatenate([src2[:EH2], src2[EH:EH + EH2]])
    src_b = jnp.concatenate([src2[EH2:EH], src2[EH + EH2:]])
    dst_a = jnp.concatenate([dst2[:EH2], dst2[EH:EH + EH2]])
    dst_b = jnp.concatenate([dst2[EH2:EH], dst2[EH + EH2:]])
    ea2 = edge_attr.reshape(EH, 2 * B)
    ea_a = ea2[:EH2]
    ea_b = ea2[EH2:]
    tt = tree_tgt_nodes.astype(jnp.int32)
    gid = graph_ids.astype(jnp.int32).reshape(N, 1)
    wix = W_i[:, :A]
    wie = W_i[:, A:]
    wox = W_o[:, :A]
    wom = W_o[:, A:]
    bo = b_o.reshape(1, H)

    zero_init = jnp.zeros((N, H), _f32)
    node_alpha = _scatter_T(tree_alpha, tt, zero_init)
    xw = _tc_xw(x, wix)
    g0_a = _gather_h(xw, src_a).reshape(2, EH2, H)
    g0_b = _gather_h(xw, src_b).reshape(2, EH2, H)
    mi_a, msg_a = _tc_init_h(g0_a, ea_a, wie)
    mi_b, msg_b = _tc_init_h(g0_b, ea_b, wie)
    for _ in range(DEPTH - 1):
        p_tab = _scatter_h(msg_a.reshape(EH, H), dst_a, node_alpha)
        g_a, s_tab = _fused_half(msg_b.reshape(EH, H), dst_b, src_a, p_tab)
        g_b = _gather_h(s_tab, src_b).reshape(2, EH2, H)
        msg_a = _tc_step_h(mi_a, g_a.reshape(2, EH2, H), msg_a, W_h)
        msg_b = _tc_step_h(mi_b, g_b, msg_b, W_h)
    p5 = _scatter_h(msg_a.reshape(EH, H), dst_a, node_alpha)
    m = _scatter_h(msg_b.reshape(EH, H), dst_b, p5)
    gr = _tc_readout(x, m, gid, wox, wom, bo)
    return gr[:G]
